# baseline (speedup 1.0000x reference)
import jax
import jax.numpy as jnp
from jax.experimental import pallas as pl
from jax.experimental.pallas import tpu as pltpu

N = 4096
CHUNK = 256
NBLK = N // CHUNK


def _fused_kernel(sr_ref, sp_ref, h0r_ref, h0p_ref,
                  w0_ref, sw0_ref, w1_ref, sw1_ref,
                  out1r_ref, out1p_ref,
                  vu_ref, vv_ref, self_r_ref, self_p_ref,
                  h1r_ref, h1p_ref):
    l = pl.program_id(0)
    s = pl.program_id(1)
    r = pl.program_id(2)
    first = jnp.logical_and(s == 0, r == 0)
    rows = pl.ds(r * CHUNK, CHUNK)

    @pl.when(jnp.logical_and(l == 0, first))
    def _init0():
        hr = h0r_ref[...]
        hp = h0p_ref[...]
        w0 = w0_ref[0]
        w1 = w0_ref[1]
        sw = sw0_ref[...]
        vu_ref[...] = jnp.concatenate(
            [jnp.dot(hr, w0, preferred_element_type=jnp.float32),
             jnp.dot(hr, w1, preferred_element_type=jnp.float32)],
            axis=1).astype(jnp.bfloat16)
        vv_ref[...] = jnp.concatenate(
            [jnp.dot(hp, w0, preferred_element_type=jnp.float32),
             jnp.dot(hp, w1, preferred_element_type=jnp.float32)],
            axis=1).astype(jnp.bfloat16)
        self_r_ref[...] = jnp.dot(hr, sw, preferred_element_type=jnp.float32)
        self_p_ref[...] = jnp.dot(hp, sw, preferred_element_type=jnp.float32)

    @pl.when(jnp.logical_and(l == 1, first))
    def _init1():
        hr = h1r_ref[...]
        hp = h1p_ref[...]
        w0 = w1_ref[0]
        w1 = w1_ref[1]
        sw = sw1_ref[...]
        vu_ref[:, :32] = jnp.concatenate(
            [jnp.dot(hr, w0, preferred_element_type=jnp.float32),
             jnp.dot(hr, w1, preferred_element_type=jnp.float32)],
            axis=1).astype(jnp.bfloat16)
        vv_ref[:, :32] = jnp.concatenate(
            [jnp.dot(hp, w0, preferred_element_type=jnp.float32),
             jnp.dot(hp, w1, preferred_element_type=jnp.float32)],
            axis=1).astype(jnp.bfloat16)
        self_r_ref[:, :32] = jnp.dot(hr, sw,
                                     preferred_element_type=jnp.float32)
        self_p_ref[:, :32] = jnp.dot(hp, sw,
                                     preferred_element_type=jnp.float32)

    sr = sr_ref[...].astype(jnp.bfloat16)
    sp = sp_ref[...].astype(jnp.bfloat16)

    def _body(l_val, s_val):
        k = 32 if l_val == 0 else 16
        lo = s_val * k
        agg_r = jnp.dot(sr, vv_ref[:, lo:lo + k],
                        preferred_element_type=jnp.float32)
        agg_p = jnp.dot(sp, vu_ref[:, lo:lo + k],
                        preferred_element_type=jnp.float32)
        h_r = jnp.maximum(agg_r + self_r_ref[rows, lo:lo + k], 0.0)
        h_p = jnp.maximum(agg_p + self_p_ref[rows, lo:lo + k], 0.0)
        if l_val == 0:
            h1r_ref[rows, lo:lo + k] = h_r
            h1p_ref[rows, lo:lo + k] = h_p
        else:
            out1r_ref[0] = h_r
            out1p_ref[0] = h_p

    for l_val in (0, 1):
        for s_val in (0, 1):
            pl.when(jnp.logical_and(l == l_val, s == s_val))(
                lambda lv=l_val, sv=s_val: _body(lv, sv))


def kernel(RNA_supports, protein_supports, RNA_inputs, protein_inputs,
           W0, W1, SW0, SW1):
    sr = RNA_supports.reshape(2 * N, N)
    sp = protein_supports.reshape(2 * N, N)
    sup_spec = pl.BlockSpec((CHUNK, N), lambda l, s, r: (s * NBLK + r, 0))
    full2 = lambda l, s, r: (0, 0)
    full3 = lambda l, s, r: (0, 0, 0)
    out_spec = pl.BlockSpec((1, CHUNK, 16), lambda l, s, r: (s, r, 0))
    o_r, o_p = pl.pallas_call(
        _fused_kernel,
        grid_spec=pltpu.PrefetchScalarGridSpec(
            num_scalar_prefetch=0,
            grid=(2, 2, NBLK),
            in_specs=[
                sup_spec,
                sup_spec,
                pl.BlockSpec((N, 128), full2),
                pl.BlockSpec((N, 128), full2),
                pl.BlockSpec((2, 128, 32), full3),
                pl.BlockSpec((128, 64), full2),
                pl.BlockSpec((2, 64, 16), full3),
                pl.BlockSpec((64, 32), full2),
            ],
            out_specs=[out_spec, out_spec],
            scratch_shapes=[
                pltpu.VMEM((N, 64), jnp.bfloat16),
                pltpu.VMEM((N, 64), jnp.bfloat16),
                pltpu.VMEM((N, 64), jnp.float32),
                pltpu.VMEM((N, 64), jnp.float32),
                pltpu.VMEM((N, 64), jnp.float32),
                pltpu.VMEM((N, 64), jnp.float32),
            ],
        ),
        out_shape=[
            jax.ShapeDtypeStruct((2, N, 16), jnp.float32),
            jax.ShapeDtypeStruct((2, N, 16), jnp.float32),
        ],
        compiler_params=pltpu.CompilerParams(
            dimension_semantics=("arbitrary", "arbitrary", "arbitrary"),
        ),
    )(sr, sp, RNA_inputs, protein_inputs, W0, SW0, W1, SW1)
    out_r = jnp.concatenate([o_r[0], o_r[1]], axis=1)
    out_p = jnp.concatenate([o_p[0], o_p[1]], axis=1)
    return (out_r, out_p)


# speedup vs baseline: 1.0438x; 1.0438x over previous
import jax
import jax.numpy as jnp
from jax.experimental import pallas as pl
from jax.experimental.pallas import tpu as pltpu

N = 4096
CHUNK = 512
NBLK = N // CHUNK


def _fused_kernel(sr_ref, sp_ref, h0r_ref, h0p_ref,
                  w0_ref, sw0_ref, w1_ref, sw1_ref,
                  out1r_ref, out1p_ref,
                  vu_ref, vv_ref, self_r_ref, self_p_ref,
                  h1r_ref, h1p_ref):
    l = pl.program_id(0)
    i = pl.program_id(1)
    first = i == 0
    rows = pl.ds((i % NBLK) * CHUNK, CHUNK)

    @pl.when(jnp.logical_and(l == 0, first))
    def _init0():
        hr = h0r_ref[...]
        hp = h0p_ref[...]
        w0 = w0_ref[0]
        w1 = w0_ref[1]
        sw = sw0_ref[...]
        vu_ref[...] = jnp.concatenate(
            [jnp.dot(hr, w0, preferred_element_type=jnp.float32),
             jnp.dot(hr, w1, preferred_element_type=jnp.float32)],
            axis=1).astype(jnp.bfloat16)
        vv_ref[...] = jnp.concatenate(
            [jnp.dot(hp, w0, preferred_element_type=jnp.float32),
             jnp.dot(hp, w1, preferred_element_type=jnp.float32)],
            axis=1).astype(jnp.bfloat16)
        self_r_ref[...] = jnp.dot(hr, sw, preferred_element_type=jnp.float32)
        self_p_ref[...] = jnp.dot(hp, sw, preferred_element_type=jnp.float32)

    @pl.when(jnp.logical_and(l == 1, first))
    def _init1():
        hr = h1r_ref[...]
        hp = h1p_ref[...]
        w0 = w1_ref[0]
        w1 = w1_ref[1]
        sw = sw1_ref[...]
        vu_ref[:, :32] = jnp.concatenate(
            [jnp.dot(hr, w0, preferred_element_type=jnp.float32),
             jnp.dot(hr, w1, preferred_element_type=jnp.float32)],
            axis=1).astype(jnp.bfloat16)
        vv_ref[:, :32] = jnp.concatenate(
            [jnp.dot(hp, w0, preferred_element_type=jnp.float32),
             jnp.dot(hp, w1, preferred_element_type=jnp.float32)],
            axis=1).astype(jnp.bfloat16)
        self_r_ref[:, :32] = jnp.dot(hr, sw,
                                     preferred_element_type=jnp.float32)
        self_p_ref[:, :32] = jnp.dot(hp, sw,
                                     preferred_element_type=jnp.float32)

    sr = sr_ref[...].astype(jnp.bfloat16)
    sp = sp_ref[...].astype(jnp.bfloat16)

    def _body(l_val, s_val):
        k = 32 if l_val == 0 else 16
        lo = s_val * k
        agg_r = jnp.dot(sr, vv_ref[:, lo:lo + k],
                        preferred_element_type=jnp.float32)
        agg_p = jnp.dot(sp, vu_ref[:, lo:lo + k],
                        preferred_element_type=jnp.float32)
        h_r = jnp.maximum(agg_r + self_r_ref[rows, lo:lo + k], 0.0)
        h_p = jnp.maximum(agg_p + self_p_ref[rows, lo:lo + k], 0.0)
        if l_val == 0:
            h1r_ref[rows, lo:lo + k] = h_r
            h1p_ref[rows, lo:lo + k] = h_p
        else:
            out1r_ref[0] = h_r
            out1p_ref[0] = h_p

    for l_val in (0, 1):
        for s_val in (0, 1):
            pl.when(jnp.logical_and(l == l_val,
                                    (i // NBLK) == s_val))(
                lambda lv=l_val, sv=s_val: _body(lv, sv))


def kernel(RNA_supports, protein_supports, RNA_inputs, protein_inputs,
           W0, W1, SW0, SW1):
    sr = RNA_supports.reshape(2 * N, N)
    sp = protein_supports.reshape(2 * N, N)
    sup_spec = pl.BlockSpec((CHUNK, N), lambda l, i: (i, 0))
    full2 = lambda l, i: (0, 0)
    full3 = lambda l, i: (0, 0, 0)
    out_spec = pl.BlockSpec((1, CHUNK, 16), lambda l, i: (i // NBLK, i % NBLK, 0))
    o_r, o_p = pl.pallas_call(
        _fused_kernel,
        grid_spec=pltpu.PrefetchScalarGridSpec(
            num_scalar_prefetch=0,
            grid=(2, 2 * NBLK),
            in_specs=[
                sup_spec,
                sup_spec,
                pl.BlockSpec((N, 128), full2),
                pl.BlockSpec((N, 128), full2),
                pl.BlockSpec((2, 128, 32), full3),
                pl.BlockSpec((128, 64), full2),
                pl.BlockSpec((2, 64, 16), full3),
                pl.BlockSpec((64, 32), full2),
            ],
            out_specs=[out_spec, out_spec],
            scratch_shapes=[
                pltpu.VMEM((N, 64), jnp.bfloat16),
                pltpu.VMEM((N, 64), jnp.bfloat16),
                pltpu.VMEM((N, 64), jnp.float32),
                pltpu.VMEM((N, 64), jnp.float32),
                pltpu.VMEM((N, 64), jnp.float32),
                pltpu.VMEM((N, 64), jnp.float32),
            ],
        ),
        out_shape=[
            jax.ShapeDtypeStruct((2, N, 16), jnp.float32),
            jax.ShapeDtypeStruct((2, N, 16), jnp.float32),
        ],
        compiler_params=pltpu.CompilerParams(
            dimension_semantics=("arbitrary", "arbitrary"),
        ),
    )(sr, sp, RNA_inputs, protein_inputs, W0, SW0, W1, SW1)
    out_r = jnp.concatenate([o_r[0], o_r[1]], axis=1)
    out_p = jnp.concatenate([o_p[0], o_p[1]], axis=1)
    return (out_r, out_p)


# speedup vs baseline: 1.0708x; 1.0258x over previous
import functools

import jax
import jax.numpy as jnp
from jax.experimental import pallas as pl
from jax.experimental.pallas import tpu as pltpu

N = 4096
BLOCK = 256


def _fused_kernel(sr_ref, sp_ref, h0r_ref, h0p_ref,
                  w0_ref, sw0_ref, w1_ref, sw1_ref,
                  out0r_ref, out0p_ref, out1r_ref, out1p_ref,
                  vu_ref, vv_ref, self_r_ref, self_p_ref,
                  h1r_ref, h1p_ref, *, block):
    l = pl.program_id(0)
    i = pl.program_id(1)
    rows = pl.ds(i * block, block)

    @pl.when(jnp.logical_and(l == 0, i == 0))
    def _init0():
        hr = h0r_ref[...]
        hp = h0p_ref[...]
        w0 = w0_ref[0]
        w1 = w0_ref[1]
        sw = sw0_ref[...]
        vu_ref[...] = jnp.concatenate(
            [jnp.dot(hr, w0, preferred_element_type=jnp.float32),
             jnp.dot(hr, w1, preferred_element_type=jnp.float32)],
            axis=1).astype(jnp.bfloat16)
        vv_ref[...] = jnp.concatenate(
            [jnp.dot(hp, w0, preferred_element_type=jnp.float32),
             jnp.dot(hp, w1, preferred_element_type=jnp.float32)],
            axis=1).astype(jnp.bfloat16)
        self_r_ref[...] = jnp.dot(hr, sw, preferred_element_type=jnp.float32)
        self_p_ref[...] = jnp.dot(hp, sw, preferred_element_type=jnp.float32)

    @pl.when(jnp.logical_and(l == 1, i == 0))
    def _init1():
        hr = h1r_ref[...]
        hp = h1p_ref[...]
        w0 = w1_ref[0]
        w1 = w1_ref[1]
        sw = sw1_ref[...]
        vu_ref[:, :32] = jnp.concatenate(
            [jnp.dot(hr, w0, preferred_element_type=jnp.float32),
             jnp.dot(hr, w1, preferred_element_type=jnp.float32)],
            axis=1).astype(jnp.bfloat16)
        vv_ref[:, :32] = jnp.concatenate(
            [jnp.dot(hp, w0, preferred_element_type=jnp.float32),
             jnp.dot(hp, w1, preferred_element_type=jnp.float32)],
            axis=1).astype(jnp.bfloat16)
        self_r_ref[:, :32] = jnp.dot(hr, sw,
                                     preferred_element_type=jnp.float32)
        self_p_ref[:, :32] = jnp.dot(hp, sw,
                                     preferred_element_type=jnp.float32)

    sr0 = sr_ref[0].astype(jnp.bfloat16)
    sr1 = sr_ref[1].astype(jnp.bfloat16)
    sp0 = sp_ref[0].astype(jnp.bfloat16)
    sp1 = sp_ref[1].astype(jnp.bfloat16)

    @pl.when(l == 0)
    def _body0():
        k = 32
        vu = vu_ref[...]
        vv = vv_ref[...]
        agg_r = jnp.concatenate(
            [jnp.dot(sr0, vv[:, :k], preferred_element_type=jnp.float32),
             jnp.dot(sr1, vv[:, k:], preferred_element_type=jnp.float32)],
            axis=1)
        agg_p = jnp.concatenate(
            [jnp.dot(sp0, vu[:, :k], preferred_element_type=jnp.float32),
             jnp.dot(sp1, vu[:, k:], preferred_element_type=jnp.float32)],
            axis=1)
        h_r = jnp.maximum(agg_r + self_r_ref[rows, :], 0.0)
        h_p = jnp.maximum(agg_p + self_p_ref[rows, :], 0.0)
        out0r_ref[...] = h_r
        out0p_ref[...] = h_p
        h1r_ref[rows, :] = h_r
        h1p_ref[rows, :] = h_p

    @pl.when(l == 1)
    def _body1():
        k = 16
        vu = vu_ref[:, :32]
        vv = vv_ref[:, :32]
        agg_r = jnp.concatenate(
            [jnp.dot(sr0, vv[:, :k], preferred_element_type=jnp.float32),
             jnp.dot(sr1, vv[:, k:], preferred_element_type=jnp.float32)],
            axis=1)
        agg_p = jnp.concatenate(
            [jnp.dot(sp0, vu[:, :k], preferred_element_type=jnp.float32),
             jnp.dot(sp1, vu[:, k:], preferred_element_type=jnp.float32)],
            axis=1)
        out1r_ref[...] = jnp.maximum(agg_r + self_r_ref[rows, :32], 0.0)
        out1p_ref[...] = jnp.maximum(agg_p + self_p_ref[rows, :32], 0.0)


def kernel(RNA_supports, protein_supports, RNA_inputs, protein_inputs,
           W0, W1, SW0, SW1):
    block = BLOCK
    nblk = N // block
    kern = functools.partial(_fused_kernel, block=block)
    sup_spec = pl.BlockSpec((2, block, N), lambda l, i: (0, i, 0))
    full2 = lambda l, i: (0, 0)
    full3 = lambda l, i: (0, 0, 0)
    out = pl.pallas_call(
        kern,
        grid_spec=pltpu.PrefetchScalarGridSpec(
            num_scalar_prefetch=0,
            grid=(2, nblk),
            in_specs=[
                sup_spec,
                sup_spec,
                pl.BlockSpec((N, 128), full2),
                pl.BlockSpec((N, 128), full2),
                pl.BlockSpec((2, 128, 32), full3),
                pl.BlockSpec((128, 64), full2),
                pl.BlockSpec((2, 64, 16), full3),
                pl.BlockSpec((64, 32), full2),
            ],
            out_specs=[
                pl.BlockSpec((block, 64), lambda l, i: (i, 0)),
                pl.BlockSpec((block, 64), lambda l, i: (i, 0)),
                pl.BlockSpec((block, 32), lambda l, i: (i, 0)),
                pl.BlockSpec((block, 32), lambda l, i: (i, 0)),
            ],
            scratch_shapes=[
                pltpu.VMEM((N, 64), jnp.bfloat16),
                pltpu.VMEM((N, 64), jnp.bfloat16),
                pltpu.VMEM((N, 64), jnp.float32),
                pltpu.VMEM((N, 64), jnp.float32),
                pltpu.VMEM((N, 64), jnp.float32),
                pltpu.VMEM((N, 64), jnp.float32),
            ],
        ),
        out_shape=[
            jax.ShapeDtypeStruct((N, 64), jnp.float32),
            jax.ShapeDtypeStruct((N, 64), jnp.float32),
            jax.ShapeDtypeStruct((N, 32), jnp.float32),
            jax.ShapeDtypeStruct((N, 32), jnp.float32),
        ],
        compiler_params=pltpu.CompilerParams(
            dimension_semantics=("arbitrary", "arbitrary"),
        ),
    )(RNA_supports, protein_supports, RNA_inputs, protein_inputs,
      W0, SW0, W1, SW1)
    return (out[2], out[3])


# speedup vs baseline: 1.0791x; 1.0078x over previous
import functools

import jax
import jax.numpy as jnp
from jax.experimental import pallas as pl
from jax.experimental.pallas import tpu as pltpu

N = 4096
BLOCK = 256


def _fused_kernel(sr_ref, sp_ref, h0r_ref, h0p_ref,
                  w0_ref, sw0_ref, w1_ref, sw1_ref,
                  out1r_ref, out1p_ref,
                  vu_ref, vv_ref, self_r_ref, self_p_ref,
                  h1r_ref, h1p_ref, *, block):
    l = pl.program_id(0)
    i = pl.program_id(1)
    rows = pl.ds(i * block, block)

    @pl.when(jnp.logical_and(l == 0, i == 0))
    def _init0():
        hr = h0r_ref[...]
        hp = h0p_ref[...]
        w0 = w0_ref[0]
        w1 = w0_ref[1]
        sw = sw0_ref[...]
        vu_ref[...] = jnp.concatenate(
            [jnp.dot(hr, w0, preferred_element_type=jnp.float32),
             jnp.dot(hr, w1, preferred_element_type=jnp.float32)],
            axis=1).astype(jnp.bfloat16)
        vv_ref[...] = jnp.concatenate(
            [jnp.dot(hp, w0, preferred_element_type=jnp.float32),
             jnp.dot(hp, w1, preferred_element_type=jnp.float32)],
            axis=1).astype(jnp.bfloat16)
        self_r_ref[...] = jnp.dot(hr, sw, preferred_element_type=jnp.float32)
        self_p_ref[...] = jnp.dot(hp, sw, preferred_element_type=jnp.float32)

    @pl.when(jnp.logical_and(l == 1, i == 0))
    def _init1():
        hr = h1r_ref[...]
        hp = h1p_ref[...]
        w0 = w1_ref[0]
        w1 = w1_ref[1]
        sw = sw1_ref[...]
        vu_ref[:, :32] = jnp.concatenate(
            [jnp.dot(hr, w0, preferred_element_type=jnp.float32),
             jnp.dot(hr, w1, preferred_element_type=jnp.float32)],
            axis=1).astype(jnp.bfloat16)
        vv_ref[:, :32] = jnp.concatenate(
            [jnp.dot(hp, w0, preferred_element_type=jnp.float32),
             jnp.dot(hp, w1, preferred_element_type=jnp.float32)],
            axis=1).astype(jnp.bfloat16)
        self_r_ref[:, :32] = jnp.dot(hr, sw,
                                     preferred_element_type=jnp.float32)
        self_p_ref[:, :32] = jnp.dot(hp, sw,
                                     preferred_element_type=jnp.float32)

    sr0 = sr_ref[0].astype(jnp.bfloat16)
    sr1 = sr_ref[1].astype(jnp.bfloat16)
    sp0 = sp_ref[0].astype(jnp.bfloat16)
    sp1 = sp_ref[1].astype(jnp.bfloat16)

    @pl.when(l == 0)
    def _body0():
        k = 32
        vu = vu_ref[...]
        vv = vv_ref[...]
        agg_r = jnp.concatenate(
            [jnp.dot(sr0, vv[:, :k], preferred_element_type=jnp.float32),
             jnp.dot(sr1, vv[:, k:], preferred_element_type=jnp.float32)],
            axis=1)
        agg_p = jnp.concatenate(
            [jnp.dot(sp0, vu[:, :k], preferred_element_type=jnp.float32),
             jnp.dot(sp1, vu[:, k:], preferred_element_type=jnp.float32)],
            axis=1)
        h1r_ref[rows, :] = jnp.maximum(agg_r + self_r_ref[rows, :], 0.0)
        h1p_ref[rows, :] = jnp.maximum(agg_p + self_p_ref[rows, :], 0.0)

    @pl.when(l == 1)
    def _body1():
        k = 16
        vu = vu_ref[:, :32]
        vv = vv_ref[:, :32]
        agg_r = jnp.concatenate(
            [jnp.dot(sr0, vv[:, :k], preferred_element_type=jnp.float32),
             jnp.dot(sr1, vv[:, k:], preferred_element_type=jnp.float32)],
            axis=1)
        agg_p = jnp.concatenate(
            [jnp.dot(sp0, vu[:, :k], preferred_element_type=jnp.float32),
             jnp.dot(sp1, vu[:, k:], preferred_element_type=jnp.float32)],
            axis=1)
        out1r_ref[...] = jnp.maximum(agg_r + self_r_ref[rows, :32], 0.0)
        out1p_ref[...] = jnp.maximum(agg_p + self_p_ref[rows, :32], 0.0)


def kernel(RNA_supports, protein_supports, RNA_inputs, protein_inputs,
           W0, W1, SW0, SW1):
    block = BLOCK
    nblk = N // block
    kern = functools.partial(_fused_kernel, block=block)
    sup_spec = pl.BlockSpec((2, block, N), lambda l, i: (0, i, 0))
    full2 = lambda l, i: (0, 0)
    full3 = lambda l, i: (0, 0, 0)
    out = pl.pallas_call(
        kern,
        grid_spec=pltpu.PrefetchScalarGridSpec(
            num_scalar_prefetch=0,
            grid=(2, nblk),
            in_specs=[
                sup_spec,
                sup_spec,
                pl.BlockSpec((N, 128), full2),
                pl.BlockSpec((N, 128), full2),
                pl.BlockSpec((2, 128, 32), full3),
                pl.BlockSpec((128, 64), full2),
                pl.BlockSpec((2, 64, 16), full3),
                pl.BlockSpec((64, 32), full2),
            ],
            out_specs=[
                pl.BlockSpec((block, 32), lambda l, i: (i, 0)),
                pl.BlockSpec((block, 32), lambda l, i: (i, 0)),
            ],
            scratch_shapes=[
                pltpu.VMEM((N, 64), jnp.bfloat16),
                pltpu.VMEM((N, 64), jnp.bfloat16),
                pltpu.VMEM((N, 64), jnp.float32),
                pltpu.VMEM((N, 64), jnp.float32),
                pltpu.VMEM((N, 64), jnp.float32),
                pltpu.VMEM((N, 64), jnp.float32),
            ],
        ),
        out_shape=[
            jax.ShapeDtypeStruct((N, 32), jnp.float32),
            jax.ShapeDtypeStruct((N, 32), jnp.float32),
        ],
        compiler_params=pltpu.CompilerParams(
            dimension_semantics=("arbitrary", "arbitrary"),
        ),
    )(RNA_supports, protein_supports, RNA_inputs, protein_inputs,
      W0, SW0, W1, SW1)
    return (out[0], out[1])


# speedup vs baseline: 1.0970x; 1.0165x over previous
import functools

import jax
import jax.numpy as jnp
from jax.experimental import pallas as pl
from jax.experimental.pallas import tpu as pltpu

N = 4096
BLOCK = 256


def _fused_kernel(sr0_ref, sr1_ref, sp0_ref, sp1_ref, h0r_ref, h0p_ref,
                  w0_ref, sw0_ref, w1_ref, sw1_ref,
                  out1r_ref, out1p_ref,
                  vu_ref, vv_ref, self_r_ref, self_p_ref,
                  h1r_ref, h1p_ref, *, block):
    l = pl.program_id(0)
    i = pl.program_id(1)
    rows = pl.ds(i * block, block)

    @pl.when(jnp.logical_and(l == 0, i == 0))
    def _init0():
        hr = h0r_ref[...]
        hp = h0p_ref[...]
        w0 = w0_ref[0]
        w1 = w0_ref[1]
        sw = sw0_ref[...]
        vu_ref[...] = jnp.concatenate(
            [jnp.dot(hr, w0, preferred_element_type=jnp.float32),
             jnp.dot(hr, w1, preferred_element_type=jnp.float32)],
            axis=1).astype(jnp.bfloat16)
        vv_ref[...] = jnp.concatenate(
            [jnp.dot(hp, w0, preferred_element_type=jnp.float32),
             jnp.dot(hp, w1, preferred_element_type=jnp.float32)],
            axis=1).astype(jnp.bfloat16)
        self_r_ref[...] = jnp.dot(hr, sw, preferred_element_type=jnp.float32)
        self_p_ref[...] = jnp.dot(hp, sw, preferred_element_type=jnp.float32)

    @pl.when(jnp.logical_and(l == 1, i == 0))
    def _init1():
        hr = h1r_ref[...]
        hp = h1p_ref[...]
        w0 = w1_ref[0]
        w1 = w1_ref[1]
        sw = sw1_ref[...]
        vu_ref[:, :32] = jnp.concatenate(
            [jnp.dot(hr, w0, preferred_element_type=jnp.float32),
             jnp.dot(hr, w1, preferred_element_type=jnp.float32)],
            axis=1).astype(jnp.bfloat16)
        vv_ref[:, :32] = jnp.concatenate(
            [jnp.dot(hp, w0, preferred_element_type=jnp.float32),
             jnp.dot(hp, w1, preferred_element_type=jnp.float32)],
            axis=1).astype(jnp.bfloat16)
        self_r_ref[:, :32] = jnp.dot(hr, sw,
                                     preferred_element_type=jnp.float32)
        self_p_ref[:, :32] = jnp.dot(hp, sw,
                                     preferred_element_type=jnp.float32)

    sr0 = sr0_ref[0].astype(jnp.bfloat16)
    sr1 = sr1_ref[0].astype(jnp.bfloat16)
    sp0 = sp0_ref[0].astype(jnp.bfloat16)
    sp1 = sp1_ref[0].astype(jnp.bfloat16)

    @pl.when(l == 0)
    def _body0():
        k = 32
        vu = vu_ref[...]
        vv = vv_ref[...]
        agg_r = jnp.concatenate(
            [jnp.dot(sr0, vv[:, :k], preferred_element_type=jnp.float32),
             jnp.dot(sr1, vv[:, k:], preferred_element_type=jnp.float32)],
            axis=1)
        agg_p = jnp.concatenate(
            [jnp.dot(sp0, vu[:, :k], preferred_element_type=jnp.float32),
             jnp.dot(sp1, vu[:, k:], preferred_element_type=jnp.float32)],
            axis=1)
        h1r_ref[rows, :] = jnp.maximum(agg_r + self_r_ref[rows, :], 0.0)
        h1p_ref[rows, :] = jnp.maximum(agg_p + self_p_ref[rows, :], 0.0)

    @pl.when(l == 1)
    def _body1():
        k = 16
        vu = vu_ref[:, :32]
        vv = vv_ref[:, :32]
        agg_r = jnp.concatenate(
            [jnp.dot(sr0, vv[:, :k], preferred_element_type=jnp.float32),
             jnp.dot(sr1, vv[:, k:], preferred_element_type=jnp.float32)],
            axis=1)
        agg_p = jnp.concatenate(
            [jnp.dot(sp0, vu[:, :k], preferred_element_type=jnp.float32),
             jnp.dot(sp1, vu[:, k:], preferred_element_type=jnp.float32)],
            axis=1)
        out1r_ref[...] = jnp.maximum(agg_r + self_r_ref[rows, :32], 0.0)
        out1p_ref[...] = jnp.maximum(agg_p + self_p_ref[rows, :32], 0.0)


def kernel(RNA_supports, protein_supports, RNA_inputs, protein_inputs,
           W0, W1, SW0, SW1):
    block = BLOCK
    nblk = N // block
    kern = functools.partial(_fused_kernel, block=block)
    sup0_spec = pl.BlockSpec((1, block, N), lambda l, i: (0, i, 0))
    sup1_spec = pl.BlockSpec((1, block, N), lambda l, i: (1, i, 0))
    full2 = lambda l, i: (0, 0)
    full3 = lambda l, i: (0, 0, 0)
    out = pl.pallas_call(
        kern,
        grid_spec=pltpu.PrefetchScalarGridSpec(
            num_scalar_prefetch=0,
            grid=(2, nblk),
            in_specs=[
                sup0_spec,
                sup1_spec,
                sup0_spec,
                sup1_spec,
                pl.BlockSpec((N, 128), full2),
                pl.BlockSpec((N, 128), full2),
                pl.BlockSpec((2, 128, 32), full3),
                pl.BlockSpec((128, 64), full2),
                pl.BlockSpec((2, 64, 16), full3),
                pl.BlockSpec((64, 32), full2),
            ],
            out_specs=[
                pl.BlockSpec((block, 32), lambda l, i: (i, 0)),
                pl.BlockSpec((block, 32), lambda l, i: (i, 0)),
            ],
            scratch_shapes=[
                pltpu.VMEM((N, 64), jnp.bfloat16),
                pltpu.VMEM((N, 64), jnp.bfloat16),
                pltpu.VMEM((N, 64), jnp.float32),
                pltpu.VMEM((N, 64), jnp.float32),
                pltpu.VMEM((N, 64), jnp.float32),
                pltpu.VMEM((N, 64), jnp.float32),
            ],
        ),
        out_shape=[
            jax.ShapeDtypeStruct((N, 32), jnp.float32),
            jax.ShapeDtypeStruct((N, 32), jnp.float32),
        ],
        compiler_params=pltpu.CompilerParams(
            dimension_semantics=("arbitrary", "arbitrary"),
        ),
    )(RNA_supports, RNA_supports, protein_supports, protein_supports,
      RNA_inputs, protein_inputs, W0, SW0, W1, SW1)
    return (out[0], out[1])
